# baseline (speedup 1.0000x reference)
import functools

import jax
import jax.numpy as jnp
from jax import lax
from jax.experimental import pallas as pl
from jax.experimental.pallas import tpu as pltpu
from jax.experimental.pallas import tpu_sc as plsc

_L = 16


@functools.lru_cache(maxsize=None)
def _build(B, V, NN, D, n_cores, n_subcores):
    NW = n_cores * n_subcores
    G = 128
    D2 = 2 * D
    per_w = B // NW
    NB = per_w // G
    assert per_w % G == 0 and D % _L == 0 and NB % 4 == 0

    mesh = plsc.VectorSubcoreMesh(core_axis_name="c", subcore_axis_name="s")

    @functools.partial(
        pl.kernel,
        out_type=jax.ShapeDtypeStruct((B, D), jnp.float32),
        mesh=mesh,
        compiler_params=pltpu.CompilerParams(use_tc_tiling_on_sc=False),
        scratch_types=[
            pltpu.VMEM((per_w,), jnp.int32),
            pltpu.VMEM((4, G, D2), jnp.float32),
            pltpu.VMEM((4, G, D), jnp.float32),
            pltpu.SemaphoreType.DMA,
            pltpu.SemaphoreType.DMA,
            pltpu.SemaphoreType.DMA,
            pltpu.SemaphoreType.DMA,
            pltpu.SemaphoreType.DMA,
            pltpu.SemaphoreType.DMA,
            pltpu.SemaphoreType.DMA,
            pltpu.SemaphoreType.DMA,
        ],
    )
    def k(ids_hbm, emb2_hbm, comp_hbm, out_hbm,
          ids_v, pairs, outb, sem_g0, sem_g1, sem_g2, sem_g3,
          sem_w0, sem_w1, sem_w2, sem_w3):
        wid = lax.axis_index("s") * n_cores + lax.axis_index("c")
        base = wid * per_w
        pltpu.sync_copy(ids_hbm.at[pl.ds(base, per_w)], ids_v)

        sems_g = (sem_g0, sem_g1, sem_g2, sem_g3)
        sems_w = (sem_w0, sem_w1, sem_w2, sem_w3)

        def fire(j, p):
            pairs_p = pairs.at[p]
            sem = sems_g[p]

            def grp(t, c2):
                id16 = ids_v[pl.ds(j * G + t * _L, _L)]
                for lane in range(_L):
                    rid = id16[lane]
                    d = rid - V
                    i = t * _L + lane

                    @pl.when(d < 0)
                    def _():
                        pltpu.async_copy(
                            emb2_hbm.at[pl.ds(lax.shift_right_logical(rid, 1), 1)],
                            pairs_p.at[pl.ds(i, 1)], sem)

                    @pl.when(d >= 0)
                    def _():
                        pltpu.async_copy(
                            comp_hbm.at[pl.ds(d, 1)],
                            pairs_p.at[pl.ds(i, 1), pl.ds(0, D)], sem)
                        pltpu.async_copy(
                            comp_hbm.at[pl.ds(d, 1)],
                            pairs_p.at[pl.ds(i, 1), pl.ds(D, D)], sem)
                return c2

            lax.fori_loop(0, G // _L, grp, 0)

        def wb_start(j, p):
            pltpu.async_copy(outb.at[p], out_hbm.at[pl.ds(base + j * G, G)], sems_w[p])

        def wb_wait(j, p):
            pltpu.make_async_copy(outb.at[p], out_hbm.at[pl.ds(base + j * G, G)], sems_w[p]).wait()

        def select(j, p):
            pairs_p = pairs.at[p]
            outb_p = outb.at[p]

            def grp(t, c2):
                id16 = ids_v[pl.ds(j * G + t * _L, _L)]
                for lane in range(_L):
                    rid = id16[lane]
                    i = t * _L + lane
                    off = jnp.bitwise_and(rid, 1) * D
                    for c in range(D // _L):
                        outb_p[i, pl.ds(c * _L, _L)] = pairs_p[i, pl.ds(off + c * _L, _L)]
                return c2

            lax.fori_loop(0, G // _L, grp, 0)

        fire(0, 0)

        def phase(j, p):
            pn = (p + 1) % 4

            @pl.when(j >= 3)
            def _():
                wb_wait(j - 3, pn)

            @pl.when(j + 1 < NB)
            def _():
                fire(j + 1, pn)

            pltpu.make_async_copy(emb2_hbm.at[pl.ds(0, G)], pairs.at[p], sems_g[p]).wait()
            select(j, p)
            wb_start(j, p)

        def step(jj, carry):
            for p in range(4):
                phase(4 * jj + p, p)
            return carry

        lax.fori_loop(0, NB // 4, step, 0)
        wb_wait(NB - 3, (NB - 3) % 4)
        wb_wait(NB - 2, (NB - 2) % 4)
        wb_wait(NB - 1, (NB - 1) % 4)

    return k


def kernel(input_ids, emb_weight, comp_weight):
    BATCH, HIST = input_ids.shape
    V, D = emb_weight.shape
    NN = comp_weight.shape[0]
    info = plsc.get_sparse_core_info()
    ids_flat = input_ids.reshape(-1).astype(jnp.int32)
    emb2 = emb_weight.reshape(V // 2, 2 * D)
    k = _build(BATCH * HIST, V, NN, D, info.num_cores, info.num_subcores)
    out = k(ids_flat, emb2, comp_weight)
    return out.reshape(BATCH, HIST, D)


# speedup vs baseline: 1.5093x; 1.5093x over previous
import functools

import jax
import jax.numpy as jnp
from jax import lax
from jax.experimental import pallas as pl
from jax.experimental.pallas import tpu as pltpu
from jax.experimental.pallas import tpu_sc as plsc

_L = 16


@functools.lru_cache(maxsize=None)
def _build(B, V, NN, D, n_cores, n_subcores):
    NW = n_cores * n_subcores
    G = 128
    per_w = B // NW
    NB = per_w // G
    assert per_w % G == 0 and D % _L == 0 and NB % 2 == 0

    mesh = plsc.VectorSubcoreMesh(core_axis_name="c", subcore_axis_name="s")

    @functools.partial(
        pl.kernel,
        out_type=jax.ShapeDtypeStruct((B, D), jnp.float32),
        mesh=mesh,
        compiler_params=pltpu.CompilerParams(use_tc_tiling_on_sc=True),
        scratch_types=[
            pltpu.VMEM((per_w,), jnp.int32),
            pltpu.VMEM((4, G, D), jnp.float32),
            pltpu.SemaphoreType.DMA,
            pltpu.SemaphoreType.DMA,
            pltpu.SemaphoreType.DMA,
            pltpu.SemaphoreType.DMA,
            pltpu.SemaphoreType.DMA,
            pltpu.SemaphoreType.DMA,
            pltpu.SemaphoreType.DMA,
            pltpu.SemaphoreType.DMA,
        ],
    )
    def k(ids_hbm, emb_hbm, comp_hbm, out_hbm,
          ids_v, rows, sem_g0, sem_g1, sem_g2, sem_g3,
          sem_w0, sem_w1, sem_w2, sem_w3):
        wid = lax.axis_index("s") * n_cores + lax.axis_index("c")
        base = wid * per_w
        pltpu.sync_copy(ids_hbm.at[pl.ds(base, per_w)], ids_v)

        sems_g = (sem_g0, sem_g1, sem_g2, sem_g3)
        sems_w = (sem_w0, sem_w1, sem_w2, sem_w3)

        def fire(j, p):
            rows_p = rows.at[p]
            sem = sems_g[p]

            def grp(t, c2):
                id16 = ids_v[pl.ds(j * G + t * _L, _L)]
                for lane in range(_L):
                    rid = id16[lane]
                    d = rid - V
                    i = t * _L + lane

                    @pl.when(d < 0)
                    def _():
                        pltpu.async_copy(
                            emb_hbm.at[pl.ds(rid, 1)],
                            rows_p.at[pl.ds(i, 1)], sem)

                    @pl.when(d >= 0)
                    def _():
                        pltpu.async_copy(
                            comp_hbm.at[pl.ds(d, 1)],
                            rows_p.at[pl.ds(i, 1)], sem)
                return c2

            lax.fori_loop(0, G // _L, grp, 0)

        def drain(p):
            pltpu.make_async_copy(emb_hbm.at[pl.ds(0, G)], rows.at[p], sems_g[p]).wait()

        def wb_start(j, p):
            pltpu.async_copy(rows.at[p], out_hbm.at[pl.ds(base + j * G, G)], sems_w[p])

        def wb_wait(j, p):
            pltpu.make_async_copy(rows.at[p], out_hbm.at[pl.ds(base + j * G, G)], sems_w[p]).wait()

        fire(0, 0)

        def phase(j, p):
            pn = (p + 1) % 4

            @pl.when(j >= 3)
            def _():
                wb_wait(j - 3, pn)

            @pl.when(j + 1 < NB)
            def _():
                fire(j + 1, pn)

            drain(p)
            wb_start(j, p)

        def step(jj, carry):
            for p in range(4):
                phase(4 * jj + p, p)
            return carry

        assert NB % 4 == 0
        lax.fori_loop(0, NB // 4, step, 0)
        wb_wait(NB - 3, (NB - 3) % 4)
        wb_wait(NB - 2, (NB - 2) % 4)
        wb_wait(NB - 1, (NB - 1) % 4)

    return k


def kernel(input_ids, emb_weight, comp_weight):
    BATCH, HIST = input_ids.shape
    V, D = emb_weight.shape
    NN = comp_weight.shape[0]
    info = plsc.get_sparse_core_info()
    ids_flat = input_ids.reshape(-1).astype(jnp.int32)
    emb_rm = lax.transpose(lax.optimization_barrier(emb_weight.T), (1, 0))
    k = _build(BATCH * HIST, V, NN, D, info.num_cores, info.num_subcores)
    out = k(ids_flat, emb_rm, comp_weight)
    return out.reshape(BATCH, HIST, D)


# speedup vs baseline: 1.5356x; 1.0174x over previous
import functools

import jax
import jax.numpy as jnp
from jax import lax
from jax.experimental import pallas as pl
from jax.experimental.pallas import tpu as pltpu
from jax.experimental.pallas import tpu_sc as plsc

_L = 16


@functools.lru_cache(maxsize=None)
def _build(BATCH, HIST, V, NN, D, n_cores, n_subcores):
    NW = n_cores * n_subcores
    B = BATCH * HIST
    G = 128
    per_w = B // NW
    NB = per_w // G
    assert B % NW == 0 and per_w % HIST == 0 and per_w % G == 0
    assert D % _L == 0 and NB % 4 == 0
    b_per_w = per_w // HIST

    mesh = plsc.VectorSubcoreMesh(core_axis_name="c", subcore_axis_name="s")

    @functools.partial(
        pl.kernel,
        out_type=jax.ShapeDtypeStruct((BATCH, HIST, D), jnp.float32),
        mesh=mesh,
        compiler_params=pltpu.CompilerParams(use_tc_tiling_on_sc=True),
        scratch_types=[
            pltpu.VMEM((per_w,), jnp.int32),
            pltpu.VMEM((4, G, D), jnp.float32),
            pltpu.SemaphoreType.DMA,
            pltpu.SemaphoreType.DMA,
            pltpu.SemaphoreType.DMA,
            pltpu.SemaphoreType.DMA,
            pltpu.SemaphoreType.DMA,
            pltpu.SemaphoreType.DMA,
            pltpu.SemaphoreType.DMA,
            pltpu.SemaphoreType.DMA,
        ],
    )
    def k(ids_hbm, emb_hbm, comp_hbm, out_hbm,
          ids_v, rows, sem_g0, sem_g1, sem_g2, sem_g3,
          sem_w0, sem_w1, sem_w2, sem_w3):
        wid = lax.axis_index("s") * n_cores + lax.axis_index("c")
        base = wid * per_w
        pltpu.sync_copy(ids_hbm.at[pl.ds(base, per_w)], ids_v)

        sems_g = (sem_g0, sem_g1, sem_g2, sem_g3)
        sems_w = (sem_w0, sem_w1, sem_w2, sem_w3)

        def fire(j, p):
            rows_p = rows.at[p]
            sem = sems_g[p]

            def grp(t, c2):
                id16 = ids_v[pl.ds(j * G + t * _L, _L)]
                for lane in range(_L):
                    rid = id16[lane]
                    d = rid - V
                    i = t * _L + lane

                    @pl.when(d < 0)
                    def _():
                        pltpu.async_copy(
                            emb_hbm.at[pl.ds(rid, 1)],
                            rows_p.at[pl.ds(i, 1)], sem)

                    @pl.when(d >= 0)
                    def _():
                        pltpu.async_copy(
                            comp_hbm.at[pl.ds(d, 1)],
                            rows_p.at[pl.ds(i, 1)], sem)
                return c2

            lax.fori_loop(0, G // _L, grp, 0)

        def drain(p):
            pltpu.make_async_copy(emb_hbm.at[pl.ds(0, G)], rows.at[p], sems_g[p]).wait()

        def wb_start(p, bh):
            b, h = bh
            rows_p = rows.at[p]
            sem = sems_w[p]
            for i in range(G):
                pltpu.async_copy(rows_p.at[pl.ds(i, 1)],
                                 out_hbm.at[b, pl.ds(h, 1)], sem)
                over = h == (HIST - 1)
                h = jnp.where(over, 0, h + 1)
                b = jnp.where(over, b + 1, b)
            return (b, h)

        def wb_wait(p):
            pltpu.make_async_copy(emb_hbm.at[pl.ds(0, G)], rows.at[p], sems_w[p]).wait()

        fire(0, 0)

        def phase(j, p, bh):
            pn = (p + 1) % 4

            @pl.when(j >= 3)
            def _():
                wb_wait(pn)

            @pl.when(j + 1 < NB)
            def _():
                fire(j + 1, pn)

            drain(p)
            return wb_start(p, bh)

        def step(jj, bh):
            for p in range(4):
                bh = phase(4 * jj + p, p, bh)
            return bh

        b0 = wid * b_per_w
        lax.fori_loop(0, NB // 4, step, (b0, 0))
        wb_wait((NB - 3) % 4)
        wb_wait((NB - 2) % 4)
        wb_wait((NB - 1) % 4)

    return k


def kernel(input_ids, emb_weight, comp_weight):
    BATCH, HIST = input_ids.shape
    V, D = emb_weight.shape
    NN = comp_weight.shape[0]
    info = plsc.get_sparse_core_info()
    ids_flat = input_ids.reshape(-1).astype(jnp.int32)
    emb_rm = lax.transpose(lax.optimization_barrier(emb_weight.T), (1, 0))
    k = _build(BATCH, HIST, V, NN, D, info.num_cores, info.num_subcores)
    out = k(ids_flat, emb_rm, comp_weight)
    out_t = lax.optimization_barrier(lax.transpose(out, (1, 2, 0)))
    return lax.transpose(out_t, (2, 0, 1))
